# baseline (speedup 1.0000x reference)
import functools

import jax
import jax.numpy as jnp
from jax import lax
from jax.experimental import pallas as pl
from jax.experimental.pallas import tpu as pltpu
from jax.experimental.pallas import tpu_sc as plsc

B, T, D = 4, 4096, 768
N = B * T
NC, NS, L = 2, 16, 16
NW = NC * NS
RPW = N // NW
NG = RPW // L
CH = 2 * L
NP = NG // 2 + 1
RS = 4
K = 2


def _sc_body(x_hbm, m_hbm, emb_hbm, out_hbm,
             mask_v, midx_v, uidx_v, emb32_v, buf_v,
             sem_m, sem_g, sem_s):
    wid = lax.axis_index("s") * NC + lax.axis_index("c")
    base = wid * RPW
    iota = lax.iota(jnp.int32, L)
    zero_v = jnp.zeros((L,), jnp.int32)

    cp1 = pltpu.async_copy(m_hbm.at[pl.ds(base, RPW)], mask_v, sem_g)
    cp2 = pltpu.async_copy(emb_hbm, emb32_v, sem_m)
    cp1.wait()
    cp2.wait()

    def buf_slot(jp):
        return buf_v.at[pl.ds((jp % RS) * CH, CH)]

    def drain_g():
        pltpu.make_async_copy(
            x_hbm.at[pl.ds(0, CH)], buf_v.at[pl.ds(0, CH)], sem_g).wait()

    def drain_s():
        pltpu.make_async_copy(
            buf_v.at[pl.ds(0, CH)], out_hbm.at[pl.ds(0, CH)], sem_s).wait()

    def drain_m():
        pltpu.make_async_copy(
            emb32_v, out_hbm.at[pl.ds(0, CH)], sem_m).wait()

    def flush_pair(jp):
        @pl.when(jp >= RS)
        def _():
            drain_s()

        pltpu.async_copy(x_hbm.at[uidx_v.at[jp]], buf_slot(jp), sem_g)

        @pl.when(jp >= K)
        def _():
            drain_g()
            pltpu.async_copy(
                buf_slot(jp - K), out_hbm.at[uidx_v.at[jp - K]], sem_s)

    def group(g, carry):
        mc0, uc0, cmv0, cuv0 = carry
        mv = mask_v[pl.ds(g * L, L)]
        mb = jnp.where(mv != 0, jnp.int32(1), jnp.int32(0))
        idx0 = base + g * L
        mc, uc, cmv, cuv = mc0, uc0, cmv0, cuv0
        cnm = zero_v
        cnu = zero_v
        mbase = mc0 & ~(L - 1)
        ubase = uc0 & ~(L - 1)
        for i in range(L):
            ii = idx0 + i
            mi = mb[i]
            pe = (mc - mbase) * mi + (mi - 1) * 100
            cmv = jnp.where(iota == pe, ii, cmv)
            cnm = jnp.where(iota == pe - L, ii, cnm)
            ui = 1 - mi
            qe = (uc - ubase) * ui + (ui - 1) * 100
            cuv = jnp.where(iota == qe, ii, cuv)
            cnu = jnp.where(iota == qe - L, ii, cnu)
            mc = mc + mi
            uc = uc + ui

        cm = (mc >> 4) - (mc0 >> 4)

        @pl.when(cm == 1)
        def _():
            c = mc0 >> 4
            midx_v[c >> 1, pl.ds((c & 1) * L, L)] = cmv

            @pl.when((c & 1) == 1)
            def _():
                pltpu.async_copy(
                    emb32_v, out_hbm.at[midx_v.at[c >> 1]], sem_m)

        cmv = cnm * cm + cmv * (1 - cm)

        cu = (uc >> 4) - (uc0 >> 4)

        @pl.when(cu == 1)
        def _():
            c = uc0 >> 4
            uidx_v[c >> 1, pl.ds((c & 1) * L, L)] = cuv

            @pl.when((c & 1) == 1)
            def _():
                flush_pair(c >> 1)

        cuv = cnu * cu + cuv * (1 - cu)
        return mc, uc, cmv, cuv

    mc, uc, cmv, cuv = lax.fori_loop(
        0, NG, group, (jnp.int32(0), jnp.int32(0), zero_v, zero_v))

    rem_m = mc & (L - 1)

    @pl.when(rem_m != 0)
    def _():
        c = mc >> 4
        padded = jnp.where(iota < rem_m, cmv, cmv[0])
        midx_v[c >> 1, pl.ds((c & 1) * L, L)] = padded

    t16m = (mc + L - 1) >> 4

    @pl.when((t16m & 1) == 1)
    def _():
        lv = midx_v[t16m >> 1, pl.ds(0, L)]
        midx_v[t16m >> 1, pl.ds(L, L)] = zero_v + lv[0]

    rem_u = uc & (L - 1)

    @pl.when(rem_u != 0)
    def _():
        c = uc >> 4
        padded = jnp.where(iota < rem_u, cuv, cuv[0])
        uidx_v[c >> 1, pl.ds((c & 1) * L, L)] = padded

    t16u = (uc + L - 1) >> 4

    @pl.when((t16u & 1) == 1)
    def _():
        lv = uidx_v[t16u >> 1, pl.ds(0, L)]
        uidx_v[t16u >> 1, pl.ds(L, L)] = zero_v + lv[0]

    npm = (t16m + 1) >> 1
    npu = (t16u + 1) >> 1

    def mtail(r, _):
        pltpu.async_copy(emb32_v, out_hbm.at[midx_v.at[r]], sem_m)
        return 0

    lax.fori_loop((mc >> 4) >> 1, npm, mtail, 0)

    def utail(jp, _):
        flush_pair(jp)
        return 0

    lax.fori_loop((uc >> 4) >> 1, npu, utail, 0)

    def stail(jp, _):
        drain_g()
        pltpu.async_copy(buf_slot(jp), out_hbm.at[uidx_v.at[jp]], sem_s)
        return 0

    lax.fori_loop(jnp.maximum(npu - K, 0), npu, stail, 0)

    def sdrain(j, _):
        drain_s()
        return 0

    lax.fori_loop(0, jnp.minimum(npu, RS), sdrain, 0)

    def mdrain(j, _):
        drain_m()
        return 0

    lax.fori_loop(0, npm, mdrain, 0)


_sc_masker = functools.partial(
    pl.kernel,
    out_type=jax.ShapeDtypeStruct((N, D), jnp.float32),
    mesh=plsc.VectorSubcoreMesh(
        core_axis_name="c", subcore_axis_name="s",
        num_cores=NC, num_subcores=NS),
    scratch_types=[
        pltpu.VMEM((RPW,), jnp.int32),
        pltpu.VMEM((NP, CH), jnp.int32),
        pltpu.VMEM((NP, CH), jnp.int32),
        pltpu.VMEM((CH, D), jnp.float32),
        pltpu.VMEM((RS * CH, D), jnp.float32),
        pltpu.SemaphoreType.DMA,
        pltpu.SemaphoreType.DMA,
        pltpu.SemaphoreType.DMA,
    ],
)(_sc_body)


def kernel(x, mask, mask_emb):
    xf = x.reshape(N, D)
    mi = mask.reshape(N).astype(jnp.int32)
    emb32 = jnp.broadcast_to(mask_emb, (CH, D))
    out = _sc_masker(xf, mi, emb32)
    return out.reshape(B, T, D)


# speedup vs baseline: 1.1276x; 1.1276x over previous
import functools

import jax
import jax.numpy as jnp
from jax import lax
from jax.experimental import pallas as pl
from jax.experimental.pallas import tpu as pltpu
from jax.experimental.pallas import tpu_sc as plsc

B, T, D = 4, 4096, 768
N = B * T
NC, NS, L = 2, 16, 16
NW = NC * NS
RPW = N // NW
NG = RPW // L
R = 9
K = 7


def _sc_body(x_hbm, m_hbm, emb_hbm, out_hbm,
             mask_v, uidx_v, emb16_v, buf_v,
             sem_m, sem_g, sem_s):
    wid = lax.axis_index("s") * NC + lax.axis_index("c")
    base = wid * RPW
    iota = lax.iota(jnp.int32, L)
    zero_v = jnp.zeros((L,), jnp.int32)

    cp1 = pltpu.async_copy(m_hbm.at[pl.ds(base, RPW)], mask_v, sem_g)
    cp2 = pltpu.async_copy(emb_hbm, emb16_v, sem_m)
    cp1.wait()
    cp2.wait()

    def buf_slot(c):
        return buf_v.at[pl.ds((c % R) * L, L)]

    def drain_g():
        pltpu.make_async_copy(
            x_hbm.at[pl.ds(0, L)], buf_v.at[pl.ds(0, L)], sem_g).wait()

    def drain_s():
        pltpu.make_async_copy(
            buf_v.at[pl.ds(0, L)], out_hbm.at[pl.ds(0, L)], sem_s).wait()

    def drain_m():
        pltpu.make_async_copy(
            emb16_v, out_hbm.at[pl.ds(0, L)], sem_m).wait()

    def flush_u(c, idx_vec):
        uidx_v[pl.ds(c * L, L)] = idx_vec

        @pl.when(c >= R)
        def _():
            drain_s()

        pltpu.async_copy(x_hbm.at[idx_vec], buf_slot(c), sem_g)

        @pl.when(c >= K)
        def _():
            drain_g()
            iv = uidx_v[pl.ds((c - K) * L, L)]
            pltpu.async_copy(buf_slot(c - K), out_hbm.at[iv], sem_s)

    def group(g, carry):
        mc0, uc0, cmv0, cuv0 = carry
        mv = mask_v[pl.ds(g * L, L)]
        mb = jnp.where(mv != 0, jnp.int32(1), jnp.int32(0))
        idx0 = base + g * L
        mc, uc, cmv, cuv = mc0, uc0, cmv0, cuv0
        cnm = zero_v
        cnu = zero_v
        mbase = mc0 & ~(L - 1)
        ubase = uc0 & ~(L - 1)
        for i in range(L):
            ii = idx0 + i
            mi = mb[i]
            pe = (mc - mbase) * mi + (mi - 1) * 100
            cmv = jnp.where(iota == pe, ii, cmv)
            cnm = jnp.where(iota == pe - L, ii, cnm)
            ui = 1 - mi
            qe = (uc - ubase) * ui + (ui - 1) * 100
            cuv = jnp.where(iota == qe, ii, cuv)
            cnu = jnp.where(iota == qe - L, ii, cnu)
            mc = mc + mi
            uc = uc + ui

        cm = (mc >> 4) - (mc0 >> 4)

        @pl.when(cm == 1)
        def _():
            pltpu.async_copy(emb16_v, out_hbm.at[cmv], sem_m)

        cmv = cnm * cm + cmv * (1 - cm)

        cu = (uc >> 4) - (uc0 >> 4)

        @pl.when(cu == 1)
        def _():
            flush_u(uc0 >> 4, cuv)

        cuv = cnu * cu + cuv * (1 - cu)
        return mc, uc, cmv, cuv

    mc, uc, cmv, cuv = lax.fori_loop(
        0, NG, group, (jnp.int32(0), jnp.int32(0), zero_v, zero_v))

    rem_m = mc & (L - 1)

    @pl.when(rem_m != 0)
    def _():
        padded = jnp.where(iota < rem_m, cmv, cmv[0])
        pltpu.async_copy(emb16_v, out_hbm.at[padded], sem_m)

    rem_u = uc & (L - 1)

    @pl.when(rem_u != 0)
    def _():
        padded = jnp.where(iota < rem_u, cuv, cuv[0])
        flush_u(uc >> 4, padded)

    nu = (uc + L - 1) >> 4
    nm = (mc + L - 1) >> 4

    def tail(j, _):
        drain_g()
        iv = uidx_v[pl.ds(j * L, L)]
        pltpu.async_copy(buf_slot(j), out_hbm.at[iv], sem_s)
        return 0

    lax.fori_loop(jnp.maximum(nu - K, 0), nu, tail, 0)

    def sdrain(j, _):
        drain_s()
        return 0

    lax.fori_loop(0, jnp.minimum(nu, R), sdrain, 0)

    def mdrain(j, _):
        drain_m()
        return 0

    lax.fori_loop(0, nm, mdrain, 0)


_sc_masker = functools.partial(
    pl.kernel,
    out_type=jax.ShapeDtypeStruct((N, D), jnp.float32),
    mesh=plsc.VectorSubcoreMesh(
        core_axis_name="c", subcore_axis_name="s",
        num_cores=NC, num_subcores=NS),
    scratch_types=[
        pltpu.VMEM((RPW,), jnp.int32),
        pltpu.VMEM((RPW,), jnp.int32),
        pltpu.VMEM((L, D), jnp.float32),
        pltpu.VMEM((R * L, D), jnp.float32),
        pltpu.SemaphoreType.DMA,
        pltpu.SemaphoreType.DMA,
        pltpu.SemaphoreType.DMA,
    ],
)(_sc_body)


def kernel(x, mask, mask_emb):
    xf = x.reshape(N, D)
    mi = mask.reshape(N).astype(jnp.int32)
    emb16 = jnp.broadcast_to(mask_emb, (L, D))
    out = _sc_masker(xf, mi, emb16)
    return out.reshape(B, T, D)


# speedup vs baseline: 1.1321x; 1.0039x over previous
import functools

import jax
import jax.numpy as jnp
from jax import lax
from jax.experimental import pallas as pl
from jax.experimental.pallas import tpu as pltpu
from jax.experimental.pallas import tpu_sc as plsc

B, T, D = 4, 4096, 768
N = B * T
NC, NS, L = 2, 16, 16
NW = NC * NS
RPW = N // NW
NG = RPW // L
R = 9
K = 7


def _sc_body(x_hbm, m_hbm, emb_hbm, out_hbm,
             mask_v, midx_v, uidx_v, emb16_v, buf_v,
             sem_m, sem_g, sem_s):
    wid = lax.axis_index("s") * NC + lax.axis_index("c")
    base = wid * RPW
    iota = lax.iota(jnp.int32, L)
    zero_v = jnp.zeros((L,), jnp.int32)

    cp1 = pltpu.async_copy(m_hbm.at[pl.ds(base, RPW)], mask_v, sem_g)
    cp2 = pltpu.async_copy(emb_hbm, emb16_v, sem_m)
    cp1.wait()
    cp2.wait()

    def buf_slot(c):
        return buf_v.at[pl.ds((c % R) * L, L)]

    def drain_g():
        pltpu.make_async_copy(
            x_hbm.at[pl.ds(0, L)], buf_v.at[pl.ds(0, L)], sem_g).wait()

    def drain_s():
        pltpu.make_async_copy(
            buf_v.at[pl.ds(0, L)], out_hbm.at[pl.ds(0, L)], sem_s).wait()

    def drain_m():
        pltpu.make_async_copy(
            emb16_v, out_hbm.at[pl.ds(0, L)], sem_m).wait()

    def flush_u(c):
        idx_vec = uidx_v[pl.ds(c * L, L)]

        @pl.when(c >= R)
        def _():
            drain_s()

        pltpu.async_copy(x_hbm.at[idx_vec], buf_slot(c), sem_g)

        @pl.when(c >= K)
        def _():
            drain_g()
            iv = uidx_v[pl.ds((c - K) * L, L)]
            pltpu.async_copy(buf_slot(c - K), out_hbm.at[iv], sem_s)

    def group(g, carry):
        mc0, uc0 = carry
        mv = mask_v[pl.ds(g * L, L)]
        mb = jnp.where(mv != 0, jnp.int32(1), jnp.int32(0))
        idx0 = base + g * L
        cml = zero_v
        cul = zero_v
        pm = jnp.int32(0)
        pu = jnp.int32(0)
        for i in range(L):
            ii = idx0 + i
            mi = mb[i]
            pe = pm * mi + (mi - 1) * 100
            cml = jnp.where(iota == pe, ii, cml)
            ui = 1 - mi
            qe = pu * ui + (ui - 1) * 100
            cul = jnp.where(iota == qe, ii, cul)
            pm = pm + mi
            pu = pu + ui
        midx_v[pl.ds(mc0, L)] = cml
        uidx_v[pl.ds(uc0, L)] = cul
        mc = mc0 + pm
        uc = uc0 + pu

        cm = (mc >> 4) - (mc0 >> 4)

        @pl.when(cm == 1)
        def _():
            iv = midx_v[pl.ds((mc0 >> 4) * L, L)]
            pltpu.async_copy(emb16_v, out_hbm.at[iv], sem_m)

        cu = (uc >> 4) - (uc0 >> 4)

        @pl.when(cu == 1)
        def _():
            flush_u(uc0 >> 4)

        return mc, uc

    mc, uc = lax.fori_loop(0, NG, group, (jnp.int32(0), jnp.int32(0)))

    rem_m = mc & (L - 1)

    @pl.when(rem_m != 0)
    def _():
        ch = midx_v[pl.ds((mc >> 4) * L, L)]
        padded = jnp.where(iota < rem_m, ch, ch[0])
        pltpu.async_copy(emb16_v, out_hbm.at[padded], sem_m)

    rem_u = uc & (L - 1)

    @pl.when(rem_u != 0)
    def _():
        ch = uidx_v[pl.ds((uc >> 4) * L, L)]
        padded = jnp.where(iota < rem_u, ch, ch[0])
        uidx_v[pl.ds((uc >> 4) * L, L)] = padded
        flush_u(uc >> 4)

    nu = (uc + L - 1) >> 4
    nm = (mc + L - 1) >> 4

    def tail(j, _):
        drain_g()
        iv = uidx_v[pl.ds(j * L, L)]
        pltpu.async_copy(buf_slot(j), out_hbm.at[iv], sem_s)
        return 0

    lax.fori_loop(jnp.maximum(nu - K, 0), nu, tail, 0)

    def sdrain(j, _):
        drain_s()
        return 0

    lax.fori_loop(0, jnp.minimum(nu, R), sdrain, 0)

    def mdrain(j, _):
        drain_m()
        return 0

    lax.fori_loop(0, nm, mdrain, 0)


_sc_masker = functools.partial(
    pl.kernel,
    out_type=jax.ShapeDtypeStruct((N, D), jnp.float32),
    mesh=plsc.VectorSubcoreMesh(
        core_axis_name="c", subcore_axis_name="s",
        num_cores=NC, num_subcores=NS),
    scratch_types=[
        pltpu.VMEM((RPW,), jnp.int32),
        pltpu.VMEM((RPW + L,), jnp.int32),
        pltpu.VMEM((RPW + L,), jnp.int32),
        pltpu.VMEM((L, D), jnp.float32),
        pltpu.VMEM((R * L, D), jnp.float32),
        pltpu.SemaphoreType.DMA,
        pltpu.SemaphoreType.DMA,
        pltpu.SemaphoreType.DMA,
    ],
)(_sc_body)


def kernel(x, mask, mask_emb):
    xf = x.reshape(N, D)
    mi = mask.reshape(N).astype(jnp.int32)
    emb16 = jnp.broadcast_to(mask_emb, (L, D))
    out = _sc_masker(xf, mi, emb16)
    return out.reshape(B, T, D)
